# baseline (speedup 1.0000x reference)
import functools
import math

import jax
import jax.numpy as jnp
from jax import lax
from jax.experimental import pallas as pl
from jax.experimental.pallas import tpu as pltpu

_SIGMA = 0.1
_SCALE = -0.5 / (_SIGMA * _SIGMA)


def _tc_body(sample_ref, loc_ref, weight_ref, out_ref):
    x = loc_ref[0]
    st = sample_ref[0, 0]
    w = weight_ref[0]
    a = x * (x - 2.0 * st[None, :])
    r = lax.broadcasted_iota(jnp.int32, (128, 8), 0) // 16
    c = lax.broadcasted_iota(jnp.int32, (128, 8), 1)
    m_ones = (r == c).astype(jnp.float32)
    t = lax.dot_general(a, m_ones, (((1,), (0,)), ((), ())),
                        preferred_element_type=jnp.float32,
                        precision=lax.Precision.HIGHEST)
    snorm = jnp.sum(st * st) * 0.125
    z = _SCALE * (t + snorm) + w
    zmax = jnp.max(z)
    lse = zmax + jnp.log(jnp.sum(jnp.exp(z - zmax)))
    d = 16
    const = -0.5 * d * math.log(2.0 * math.pi * _SIGMA * _SIGMA)
    out_ref[0, 0, :] = jnp.full((128,), lse + const, jnp.float32)


def kernel(sample, loc, weight):
    B, N, D = loc.shape
    loc2 = loc.reshape(B, N // 8, 8 * D)
    w2 = weight.reshape(B, N // 8, 8)
    st = jnp.tile(sample, (1, 8)).reshape(B, 1, 8 * D)
    out = pl.pallas_call(
        _tc_body,
        grid=(B,),
        in_specs=[
            pl.BlockSpec((1, 1, 8 * D), lambda b: (b, 0, 0)),
            pl.BlockSpec((1, N // 8, 8 * D), lambda b: (b, 0, 0)),
            pl.BlockSpec((1, N // 8, 8), lambda b: (b, 0, 0)),
        ],
        out_specs=pl.BlockSpec((1, 1, 128), lambda b: (b, 0, 0)),
        out_shape=jax.ShapeDtypeStruct((B, 1, 128), jnp.float32),
    )(st, loc2, w2)
    return out[:, 0, 0]
